# initial kernel scaffold (unmeasured)
import jax
import jax.numpy as jnp
from jax import lax
from jax.experimental import pallas as pl
from jax.experimental.pallas import tpu as pltpu

B = 32
H = 16
D = 128
BS = 32
NB_LOCAL = 256
NBT = 256
Z = 4
PC = 32
CK = PC * BS
NC = NB_LOCAL // PC
HD = H * D
NEG = -1e30
SCALE = D ** -0.5


def _compute_body(q_ref, k_ref, v_ref, bt_ref, lens_ref, o_ref, m_ref, l_ref):
    c = pl.program_id(0)
    my_z = lax.axis_index("z")

    @pl.when(c == 0)
    def _init():
        m_ref[:, :] = jnp.full((B, H), NEG, jnp.float32)
        l_ref[:, :] = jnp.zeros((B, H), jnp.float32)
        o_ref[:, :] = jnp.zeros((B, HD), jnp.float32)

    pages = my_z * NB_LOCAL + c * PC + lax.broadcasted_iota(
        jnp.int32, (B, PC, NBT), 1
    )
    jidx = lax.broadcasted_iota(jnp.int32, (B, PC, NBT), 2)
    hit = (bt_ref[:, :][:, None, :] == pages) & (jidx < lens_ref[:, :][:, :, None])
    wc = jnp.sum(hit.astype(jnp.float32), axis=2)

    kk = lax.broadcasted_iota(jnp.int32, (PC, CK), 1) // BS
    pp = lax.broadcasted_iota(jnp.int32, (PC, CK), 0)
    expand = (kk == pp).astype(jnp.float32)
    wk = lax.dot_general(
        wc, expand, (((1,), (0,)), ((), ())), preferred_element_type=jnp.float32
    )
    valid = wk > 0.0

    for h in range(H):
        sl = slice(h * D, (h + 1) * D)
        qh = q_ref[:, sl]
        kh = k_ref[:, sl]
        s = lax.dot_general(
            qh, kh, (((1,), (1,)), ((), ())), preferred_element_type=jnp.float32
        ) * SCALE
        s = jnp.where(valid, s, NEG)
        m_prev = m_ref[:, h:h + 1]
        m_new = jnp.maximum(m_prev, jnp.max(s, axis=1, keepdims=True))
        alpha = jnp.exp(m_prev - m_new)
        p = wk * jnp.exp(s - m_new)
        l_ref[:, h:h + 1] = l_ref[:, h:h + 1] * alpha + jnp.sum(
            p, axis=1, keepdims=True
        )
        vh = v_ref[:, sl]
        o_ref[:, sl] = o_ref[:, sl] * alpha + lax.dot_general(
            p, vh, (((1,), (0,)), ((), ())), preferred_element_type=jnp.float32
        )
        m_ref[:, h:h + 1] = m_new


def _merge_body(o_ref, m_ref, l_ref, out_ref, obuf, mbuf, lbuf, send_sems, recv_sems):
    my_x = lax.axis_index("x")
    my_y = lax.axis_index("y")
    my_z = lax.axis_index("z")

    barrier = pltpu.get_barrier_semaphore()
    for d in (1, 2, 3):
        pl.semaphore_signal(
            barrier,
            inc=1,
            device_id=(my_x, my_y, (my_z + d) % Z),
            device_id_type=pl.DeviceIdType.MESH,
        )
    pl.semaphore_wait(barrier, 3)

    rdmas = []
    for di, d in enumerate((1, 2, 3)):
        tgt = (my_x, my_y, (my_z + d) % Z)
        for ai, (src, buf) in enumerate(
            ((o_ref, obuf), (m_ref, mbuf), (l_ref, lbuf))
        ):
            r = pltpu.make_async_remote_copy(
                src_ref=src,
                dst_ref=buf.at[di],
                send_sem=send_sems.at[di, ai],
                recv_sem=recv_sems.at[di, ai],
                device_id=tgt,
                device_id_type=pl.DeviceIdType.MESH,
            )
            r.start()
            rdmas.append(r)
    for r in rdmas:
        r.wait()

    m_all = m_ref[:, :]
    for di in range(3):
        m_all = jnp.maximum(m_all, mbuf[di, :, :])
    coefs = [jnp.exp(m_ref[:, :] - m_all)]
    l_all = l_ref[:, :] * coefs[0]
    for di in range(3):
        cdi = jnp.exp(mbuf[di, :, :] - m_all)
        l_all = l_all + lbuf[di, :, :] * cdi
        coefs.append(cdi)
    for h in range(H):
        sl = slice(h * D, (h + 1) * D)
        acc = o_ref[:, sl] * coefs[0][:, h:h + 1]
        for di in range(3):
            acc = acc + obuf[di, :, sl] * coefs[di + 1][:, h:h + 1]
        out_ref[:, sl] = acc / l_all[:, h:h + 1]


def kernel(Q, K, V, bt, lens):
    q2 = Q.reshape(B, HD)
    k2 = K.reshape(NB_LOCAL * BS, HD)
    v2 = V.reshape(NB_LOCAL * BS, HD)
    lens2 = lens.reshape(B, 1)

    o_part, m_part, l_part = pl.pallas_call(
        _compute_body,
        grid=(NC,),
        in_specs=[
            pl.BlockSpec((B, HD), lambda c: (0, 0)),
            pl.BlockSpec((CK, HD), lambda c: (c, 0)),
            pl.BlockSpec((CK, HD), lambda c: (c, 0)),
            pl.BlockSpec((B, NBT), lambda c: (0, 0)),
            pl.BlockSpec((B, 1), lambda c: (0, 0)),
        ],
        out_specs=[
            pl.BlockSpec((B, HD), lambda c: (0, 0)),
            pl.BlockSpec((B, H), lambda c: (0, 0)),
            pl.BlockSpec((B, H), lambda c: (0, 0)),
        ],
        out_shape=[
            jax.ShapeDtypeStruct((B, HD), jnp.float32),
            jax.ShapeDtypeStruct((B, H), jnp.float32),
            jax.ShapeDtypeStruct((B, H), jnp.float32),
        ],
        compiler_params=pltpu.CompilerParams(
            dimension_semantics=("arbitrary",)
        ),
    )(q2, k2, v2, bt, lens2)

    out2 = pl.pallas_call(
        _merge_body,
        in_specs=[pl.BlockSpec(memory_space=pltpu.VMEM)] * 3,
        out_specs=pl.BlockSpec(memory_space=pltpu.VMEM),
        out_shape=jax.ShapeDtypeStruct((B, HD), jnp.float32),
        scratch_shapes=[
            pltpu.VMEM((3, B, HD), jnp.float32),
            pltpu.VMEM((3, B, H), jnp.float32),
            pltpu.VMEM((3, B, H), jnp.float32),
            pltpu.SemaphoreType.DMA((3, 3)),
            pltpu.SemaphoreType.DMA((3, 3)),
        ],
        compiler_params=pltpu.CompilerParams(collective_id=0),
    )(o_part, m_part, l_part)

    return out2.reshape(B, 1, H, D)


# baseline (device time: 207877 ns/iter reference)
import jax
import jax.numpy as jnp
from jax import lax
from jax.experimental import pallas as pl
from jax.experimental.pallas import tpu as pltpu

B = 32
H = 16
D = 128
BS = 32
NB_LOCAL = 256
NBT = 256
Z = 4
PC = 16
CK = PC * BS
NC = NB_LOCAL // PC
HD = H * D
NEG = -1e30
SCALE = D ** -0.5


def _compute_body(q_ref, k_ref, v_ref, bt_ref, lens_ref, o_ref, m_ref, l_ref):
    c = pl.program_id(0)
    my_z = lax.axis_index("z")

    @pl.when(c == 0)
    def _init():
        m_ref[:, :] = jnp.full((B, H), NEG, jnp.float32)
        l_ref[:, :] = jnp.zeros((B, H), jnp.float32)
        o_ref[:, :] = jnp.zeros((B, HD), jnp.float32)

    pages = my_z * NB_LOCAL + c * PC + lax.broadcasted_iota(
        jnp.int32, (B, PC, NBT), 1
    )
    jidx = lax.broadcasted_iota(jnp.int32, (B, PC, NBT), 2)
    hit = (bt_ref[:, :][:, None, :] == pages) & (jidx < lens_ref[:, :][:, :, None])
    wc = jnp.sum(hit.astype(jnp.float32), axis=2)

    kk = lax.broadcasted_iota(jnp.int32, (PC, CK), 1) // BS
    pp = lax.broadcasted_iota(jnp.int32, (PC, CK), 0)
    expand = (kk == pp).astype(jnp.float32)
    wk = lax.dot_general(
        wc, expand, (((1,), (0,)), ((), ())), preferred_element_type=jnp.float32
    )
    valid = wk > 0.0

    for h in range(H):
        sl = slice(h * D, (h + 1) * D)
        qh = q_ref[:, sl]
        kh = k_ref[:, sl]
        s = lax.dot_general(
            qh, kh, (((1,), (1,)), ((), ())), preferred_element_type=jnp.float32
        ) * SCALE
        s = jnp.where(valid, s, NEG)
        m_prev = m_ref[:, h:h + 1]
        m_new = jnp.maximum(m_prev, jnp.max(s, axis=1, keepdims=True))
        alpha = jnp.exp(m_prev - m_new)
        p = wk * jnp.exp(s - m_new)
        l_ref[:, h:h + 1] = l_ref[:, h:h + 1] * alpha + jnp.sum(
            p, axis=1, keepdims=True
        )
        vh = v_ref[:, sl]
        o_ref[:, sl] = o_ref[:, sl] * alpha + lax.dot_general(
            p, vh, (((1,), (0,)), ((), ())), preferred_element_type=jnp.float32
        )
        m_ref[:, h:h + 1] = m_new


def _merge_body(o_ref, m_ref, l_ref, out_ref, obuf, mbuf, lbuf, send_sems, recv_sems):
    my_x = lax.axis_index("x")
    my_y = lax.axis_index("y")
    my_z = lax.axis_index("z")

    barrier = pltpu.get_barrier_semaphore()
    for d in (1, 2, 3):
        pl.semaphore_signal(
            barrier,
            inc=1,
            device_id=(my_x, my_y, (my_z + d) % Z),
            device_id_type=pl.DeviceIdType.MESH,
        )
    pl.semaphore_wait(barrier, 3)

    rdmas = []
    for di, d in enumerate((1, 2, 3)):
        tgt = (my_x, my_y, (my_z + d) % Z)
        for ai, (src, buf) in enumerate(
            ((o_ref, obuf), (m_ref, mbuf), (l_ref, lbuf))
        ):
            r = pltpu.make_async_remote_copy(
                src_ref=src,
                dst_ref=buf.at[di],
                send_sem=send_sems.at[di, ai],
                recv_sem=recv_sems.at[di, ai],
                device_id=tgt,
                device_id_type=pl.DeviceIdType.MESH,
            )
            r.start()
            rdmas.append(r)
    for r in rdmas:
        r.wait()

    m_all = m_ref[:, :]
    for di in range(3):
        m_all = jnp.maximum(m_all, mbuf[di, :, :])
    coefs = [jnp.exp(m_ref[:, :] - m_all)]
    l_all = l_ref[:, :] * coefs[0]
    for di in range(3):
        cdi = jnp.exp(mbuf[di, :, :] - m_all)
        l_all = l_all + lbuf[di, :, :] * cdi
        coefs.append(cdi)
    for h in range(H):
        sl = slice(h * D, (h + 1) * D)
        acc = o_ref[:, sl] * coefs[0][:, h:h + 1]
        for di in range(3):
            acc = acc + obuf[di, :, sl] * coefs[di + 1][:, h:h + 1]
        out_ref[:, sl] = acc / l_all[:, h:h + 1]


def kernel(Q, K, V, bt, lens):
    q2 = Q.reshape(B, HD)
    k2 = K.reshape(NB_LOCAL * BS, HD)
    v2 = V.reshape(NB_LOCAL * BS, HD)
    lens2 = lens.reshape(B, 1)

    o_part, m_part, l_part = pl.pallas_call(
        _compute_body,
        grid=(NC,),
        in_specs=[
            pl.BlockSpec((B, HD), lambda c: (0, 0)),
            pl.BlockSpec((CK, HD), lambda c: (c, 0)),
            pl.BlockSpec((CK, HD), lambda c: (c, 0)),
            pl.BlockSpec((B, NBT), lambda c: (0, 0)),
            pl.BlockSpec((B, 1), lambda c: (0, 0)),
        ],
        out_specs=[
            pl.BlockSpec((B, HD), lambda c: (0, 0)),
            pl.BlockSpec((B, H), lambda c: (0, 0)),
            pl.BlockSpec((B, H), lambda c: (0, 0)),
        ],
        out_shape=[
            jax.ShapeDtypeStruct((B, HD), jnp.float32),
            jax.ShapeDtypeStruct((B, H), jnp.float32),
            jax.ShapeDtypeStruct((B, H), jnp.float32),
        ],
        compiler_params=pltpu.CompilerParams(
            dimension_semantics=("arbitrary",)
        ),
    )(q2, k2, v2, bt, lens2)

    out2 = pl.pallas_call(
        _merge_body,
        in_specs=[pl.BlockSpec(memory_space=pltpu.VMEM)] * 3,
        out_specs=pl.BlockSpec(memory_space=pltpu.VMEM),
        out_shape=jax.ShapeDtypeStruct((B, HD), jnp.float32),
        scratch_shapes=[
            pltpu.VMEM((3, B, HD), jnp.float32),
            pltpu.VMEM((3, B, H), jnp.float32),
            pltpu.VMEM((3, B, H), jnp.float32),
            pltpu.SemaphoreType.DMA((3, 3)),
            pltpu.SemaphoreType.DMA((3, 3)),
        ],
        compiler_params=pltpu.CompilerParams(collective_id=0),
    )(o_part, m_part, l_part)

    return out2.reshape(B, 1, H, D)


# device time: 91488 ns/iter; 2.2722x vs baseline; 2.2722x over previous
import jax
import jax.numpy as jnp
from jax import lax
from jax.experimental import pallas as pl
from jax.experimental.pallas import tpu as pltpu

B = 32
H = 16
D = 128
BS = 32
NB_LOCAL = 256
NBT = 256
Z = 4
PC = 32
CK = PC * BS
NC = NB_LOCAL // PC
HD = H * D
NEG = -1e30
SCALE = D ** -0.5


def _body(
    q_ref, k_ref, v_ref, bt_ref, lens_ref, out_ref,
    o16, m_s, l_s, oacc, btm, obuf, mbuf, lbuf, send_sems, recv_sems,
):
    c = pl.program_id(0)
    my_x = lax.axis_index("x")
    my_y = lax.axis_index("y")
    my_z = lax.axis_index("z")

    @pl.when(c == 0)
    def _init():
        barrier = pltpu.get_barrier_semaphore()
        for d in (1, 2, 3):
            pl.semaphore_signal(
                barrier,
                inc=1,
                device_id=(my_x, my_y, (my_z + d) % Z),
                device_id_type=pl.DeviceIdType.MESH,
            )
        pl.semaphore_wait(barrier, 3)
        m_s[:, :] = jnp.full((B, H), NEG, jnp.float32)
        l_s[:, :] = jnp.zeros((B, H), jnp.float32)
        oacc[:, :] = jnp.zeros((B, HD), jnp.float32)
        jidx = lax.broadcasted_iota(jnp.int32, (B, NBT), 1)
        btm[:, :] = jnp.where(jidx < lens_ref[:, :], bt_ref[:, :], -1)

    pages = my_z * NB_LOCAL + c * PC + lax.broadcasted_iota(
        jnp.int32, (B, PC, NBT), 1
    )
    hit = btm[:, :][:, None, :] == pages
    wc = jnp.sum(hit.astype(jnp.float32), axis=2)

    kk = lax.broadcasted_iota(jnp.int32, (PC, CK), 1) // BS
    pp = lax.broadcasted_iota(jnp.int32, (PC, CK), 0)
    expand = (kk == pp).astype(jnp.float32)
    wk = lax.dot_general(
        wc, expand, (((1,), (0,)), ((), ())), preferred_element_type=jnp.float32
    )

    kt = jnp.transpose(k_ref[...], (2, 0, 1, 3))
    vt = jnp.transpose(v_ref[...], (2, 0, 1, 3))
    for h in range(H):
        sl = slice(h * D, (h + 1) * D)
        qh = q_ref[:, 0, h, :] * SCALE
        kh = kt[h].reshape(CK, D)
        s = lax.dot_general(
            qh, kh, (((1,), (1,)), ((), ())), preferred_element_type=jnp.float32
        )
        m_prev = m_s[:, h:h + 1]
        m_new = jnp.maximum(m_prev, jnp.max(s, axis=1, keepdims=True))
        alpha = jnp.exp(m_prev - m_new)
        p = wk * jnp.exp(s - m_new)
        l_s[:, h:h + 1] = l_s[:, h:h + 1] * alpha + jnp.sum(
            p, axis=1, keepdims=True
        )
        vh = vt[h].reshape(CK, D)
        oacc[:, sl] = oacc[:, sl] * alpha + lax.dot_general(
            p, vh, (((1,), (0,)), ((), ())), preferred_element_type=jnp.float32
        )
        m_s[:, h:h + 1] = m_new

    @pl.when(c == NC - 1)
    def _exchange_and_merge():
        o16[:, :] = oacc[:, :].astype(jnp.bfloat16)

        rdmas = []
        for di, d in enumerate((1, 2, 3)):
            tgt = (my_x, my_y, (my_z + d) % Z)
            for ai, (src, buf) in enumerate(
                ((o16, obuf), (m_s, mbuf), (l_s, lbuf))
            ):
                r = pltpu.make_async_remote_copy(
                    src_ref=src,
                    dst_ref=buf.at[di],
                    send_sem=send_sems.at[di, ai],
                    recv_sem=recv_sems.at[di, ai],
                    device_id=tgt,
                    device_id_type=pl.DeviceIdType.MESH,
                )
                r.start()
                rdmas.append(r)
        for r in rdmas:
            r.wait()

        m_all = m_s[:, :]
        for di in range(3):
            m_all = jnp.maximum(m_all, mbuf[di, :, :])
        coefs = [jnp.exp(m_s[:, :] - m_all)]
        l_all = l_s[:, :] * coefs[0]
        for di in range(3):
            cdi = jnp.exp(mbuf[di, :, :] - m_all)
            l_all = l_all + lbuf[di, :, :] * cdi
            coefs.append(cdi)
        for h in range(H):
            sl = slice(h * D, (h + 1) * D)
            acc = oacc[:, sl] * coefs[0][:, h:h + 1]
            for di in range(3):
                acc = acc + obuf[di, :, sl].astype(jnp.float32) * coefs[
                    di + 1
                ][:, h:h + 1]
            out_ref[:, 0, h, :] = acc / l_all[:, h:h + 1]


def kernel(Q, K, V, bt, lens):
    lens2 = lens.reshape(B, 1)

    return pl.pallas_call(
        _body,
        grid=(NC,),
        in_specs=[
            pl.BlockSpec((B, 1, H, D), lambda c: (0, 0, 0, 0)),
            pl.BlockSpec((PC, BS, H, D), lambda c: (c, 0, 0, 0)),
            pl.BlockSpec((PC, BS, H, D), lambda c: (c, 0, 0, 0)),
            pl.BlockSpec((B, NBT), lambda c: (0, 0)),
            pl.BlockSpec((B, 1), lambda c: (0, 0)),
        ],
        out_specs=pl.BlockSpec((B, 1, H, D), lambda c: (0, 0, 0, 0)),
        out_shape=jax.ShapeDtypeStruct((B, 1, H, D), jnp.float32),
        scratch_shapes=[
            pltpu.VMEM((B, HD), jnp.bfloat16),
            pltpu.VMEM((B, H), jnp.float32),
            pltpu.VMEM((B, H), jnp.float32),
            pltpu.VMEM((B, HD), jnp.float32),
            pltpu.VMEM((B, NBT), jnp.int32),
            pltpu.VMEM((3, B, HD), jnp.bfloat16),
            pltpu.VMEM((3, B, H), jnp.float32),
            pltpu.VMEM((3, B, H), jnp.float32),
            pltpu.SemaphoreType.DMA((3, 3)),
            pltpu.SemaphoreType.DMA((3, 3)),
        ],
        compiler_params=pltpu.CompilerParams(
            dimension_semantics=("arbitrary",),
            vmem_limit_bytes=80 * 1024 * 1024,
            collective_id=0,
        ),
    )(Q, K, V, bt, lens2)
